# speedup vs baseline: 1.7826x; 1.7826x over previous
import jax
import jax.numpy as jnp
from jax.experimental import pallas as pl
from jax.experimental.pallas import tpu as pltpu

_NUM_ROUTED = 8
_NUM_TOTAL = 10
_D_MODEL = 1024
_D_FF = 512
_N_TOKENS = 2048
_SCALE = 2.5
_TBLK = 256
_NTB = _N_TOKENS // _TBLK
_LANES = 128


def _dense_body(wrt_ref, bias_ref, x_ref, wg_ref, wu_ref, wd_ref, out_ref, comb_ref):
    e = pl.program_id(0)
    t = pl.program_id(1)

    @pl.when((e == 0) & (t == 0))
    def _router():
        logits = jnp.dot(x_ref[:], wrt_ref[:], preferred_element_type=jnp.float32)
        lane = jax.lax.broadcasted_iota(jnp.int32, logits.shape, 1)
        valid = lane < _NUM_TOTAL
        neg = jnp.float32(-1e30)
        lmask = jnp.where(valid, logits, neg)
        m = jnp.max(lmask, axis=-1, keepdims=True)
        p = jnp.where(valid, jnp.exp(lmask - m), 0.0)
        s = jnp.sum(p, axis=-1, keepdims=True)
        scores = p / s
        sel = jnp.where(valid, scores + bias_ref[:], neg)
        m1 = jnp.max(sel, axis=-1, keepdims=True)
        i1 = jnp.min(jnp.where(sel == m1, lane, _LANES), axis=-1, keepdims=True)
        w1 = jnp.sum(jnp.where(lane == i1, scores, 0.0), axis=-1, keepdims=True)
        sel2 = jnp.where(lane == i1, neg, sel)
        m2 = jnp.max(sel2, axis=-1, keepdims=True)
        i2 = jnp.min(jnp.where(sel2 == m2, lane, _LANES), axis=-1, keepdims=True)
        w2 = jnp.sum(jnp.where(lane == i2, scores, 0.0), axis=-1, keepdims=True)
        comb_ref[:] = _SCALE * (
            w1 * (lane == i1).astype(jnp.float32)
            + w2 * (lane == i2).astype(jnp.float32)
        )

    rows = pl.ds(t * _TBLK, _TBLK)
    xb = x_ref[rows, :]
    g = jnp.dot(xb, wg_ref[0], preferred_element_type=jnp.float32)
    u = jnp.dot(xb, wu_ref[0], preferred_element_type=jnp.float32)
    h = g * jax.nn.sigmoid(g) * u
    y = jnp.dot(h, wd_ref[0], preferred_element_type=jnp.float32)
    cb = comb_ref[rows, :]
    lane = jax.lax.broadcasted_iota(jnp.int32, cb.shape, 1)
    ce = jnp.sum(jnp.where(lane == e, cb, 0.0), axis=-1, keepdims=True)

    @pl.when(e == 0)
    def _init():
        out_ref[rows, :] = ce * y

    @pl.when(e > 0)
    def _acc():
        out_ref[rows, :] = out_ref[rows, :] + ce * y

    @pl.when(e == _NUM_ROUTED - 1)
    def _zero_experts():
        zw = jnp.sum(jnp.where(lane >= _NUM_ROUTED, cb, 0.0), axis=-1, keepdims=True)
        out_ref[rows, :] = out_ref[rows, :] + zw * xb


def _moe_dense(x, wrt, bias_pad, w_gate, w_up, w_down, interpret=False):
    return pl.pallas_call(
        _dense_body,
        grid=(_NUM_ROUTED, _NTB),
        in_specs=[
            pl.BlockSpec((_D_MODEL, _LANES), lambda e, t: (0, 0)),
            pl.BlockSpec((1, _LANES), lambda e, t: (0, 0)),
            pl.BlockSpec((_N_TOKENS, _D_MODEL), lambda e, t: (0, 0)),
            pl.BlockSpec((1, _D_MODEL, _D_FF), lambda e, t: (e, 0, 0)),
            pl.BlockSpec((1, _D_MODEL, _D_FF), lambda e, t: (e, 0, 0)),
            pl.BlockSpec((1, _D_FF, _D_MODEL), lambda e, t: (e, 0, 0)),
        ],
        out_specs=pl.BlockSpec((_N_TOKENS, _D_MODEL), lambda e, t: (0, 0)),
        out_shape=jax.ShapeDtypeStruct((_N_TOKENS, _D_MODEL), jnp.float32),
        scratch_shapes=[pltpu.VMEM((_N_TOKENS, _LANES), jnp.float32)],
        interpret=interpret,
    )(wrt, bias_pad, x, w_gate, w_up, w_down)


def kernel(hidden_states, num_global_tokens, max_num_tokens_per_gpu,
           router_weight, correction_bias, w_gate, w_up, w_down):
    x = hidden_states.astype(jnp.float32)
    wrt = jnp.zeros((_D_MODEL, _LANES), jnp.float32).at[:, :_NUM_TOTAL].set(
        router_weight.T.astype(jnp.float32))
    bias_pad = jnp.zeros((1, _LANES), jnp.float32).at[0, :_NUM_TOTAL].set(
        correction_bias.astype(jnp.float32))
    return _moe_dense(x, wrt, bias_pad, w_gate, w_up, w_down)
